# Initial kernel scaffold; baseline (speedup 1.0000x reference)
import jax
import jax.numpy as jnp
from jax import lax
from jax.experimental import pallas as pl
from jax.experimental.pallas import tpu as pltpu
from jax.experimental.pallas import tpu_sc as plsc

N_NODES = 10000
N_EDGES = 160000
D_FEAT = 256
HALF = 128

NC = 2
NS = 16
CHUNK = 80
EDGES_PER_TILE = N_EDGES // NS
N_CHUNKS = EDGES_PER_TILE // CHUNK
ROWS_PER_TILE = N_NODES // NS


def _pq_body(nodes_ref, w_ref, out_ref):
    out_ref[0, 0] = jnp.dot(nodes_ref[...], w_ref[0],
                            preferred_element_type=jnp.float32)


def _compute_pq(nodes, we_sd):
    blk = 2000
    grid = (N_NODES // blk, 2, 2)
    return pl.pallas_call(
        _pq_body,
        grid=grid,
        in_specs=[
            pl.BlockSpec((blk, D_FEAT), lambda i, w, c: (i, 0)),
            pl.BlockSpec((1, D_FEAT, HALF), lambda i, w, c: (w, 0, c)),
        ],
        out_specs=pl.BlockSpec((1, 1, blk, HALF), lambda i, w, c: (w, c, i, 0)),
        out_shape=jax.ShapeDtypeStruct((2, 2, N_NODES, HALF), jnp.float32),
    )(nodes, we_sd)


def _r_body(e_ref, w_ref, b_ref, out_ref):
    out_ref[0] = (jnp.dot(e_ref[...], w_ref[...],
                          preferred_element_type=jnp.float32)
                  + b_ref[...][None, :])


def _compute_r(edges, we_e, b_e):
    blk = 4000
    grid = (N_EDGES // blk, 2)
    return pl.pallas_call(
        _r_body,
        grid=grid,
        in_specs=[
            pl.BlockSpec((blk, 16), lambda j, c: (j, 0)),
            pl.BlockSpec((16, HALF), lambda j, c: (0, c)),
            pl.BlockSpec((HALF,), lambda j, c: (c,)),
        ],
        out_specs=pl.BlockSpec((1, blk, HALF), lambda j, c: (c, j, 0)),
        out_shape=jax.ShapeDtypeStruct((2, N_EDGES, HALF), jnp.float32),
    )(edges, we_e, b_e)


def _node_body(nodes_ref, agg_ref, wn1_ref, wn2_ref, b_ref, out_ref):
    acc = jnp.dot(nodes_ref[...], wn1_ref[...],
                  preferred_element_type=jnp.float32)
    acc += jnp.dot(agg_ref[0], wn2_ref[0], preferred_element_type=jnp.float32)
    acc += jnp.dot(agg_ref[1], wn2_ref[1], preferred_element_type=jnp.float32)
    out_ref[...] = jnp.maximum(acc + b_ref[...][None, :], 0.0)


def _node_update(nodes, agg, wn1, wn2r, b_n):
    blk = 2000
    grid = (N_NODES // blk, 2)
    return pl.pallas_call(
        _node_body,
        grid=grid,
        in_specs=[
            pl.BlockSpec((blk, D_FEAT), lambda i, h: (i, 0)),
            pl.BlockSpec((2, blk, HALF), lambda i, h: (0, i, 0)),
            pl.BlockSpec((D_FEAT, HALF), lambda i, h: (0, h)),
            pl.BlockSpec((2, HALF, HALF), lambda i, h: (0, 0, h)),
            pl.BlockSpec((HALF,), lambda i, h: (h,)),
        ],
        out_specs=pl.BlockSpec((blk, HALF), lambda i, h: (i, h)),
        out_shape=jax.ShapeDtypeStruct((N_NODES, D_FEAT), jnp.float32),
    )(nodes, agg, wn1, wn2r, b_n)


def _sc_body(src_hbm, dst_hbm, pf_hbm, qf_hbm, r3_hbm, z_hbm,
             enew_hbm, agg_hbm,
             sv, dv, pv, qv, pbuf, qbuf, rbuf, ebuf, acc, sem_p, sem_q):
    c = lax.axis_index("c")
    s = lax.axis_index("s")

    pltpu.sync_copy(z_hbm.at[pl.ds(s * ROWS_PER_TILE, ROWS_PER_TILE)],
                    acc.at[pl.ds(s * ROWS_PER_TILE, ROWS_PER_TILE)])
    plsc.subcore_barrier()

    off = c * N_NODES

    def chunk(i, carry):
        base = s * EDGES_PER_TILE + i * CHUNK
        pltpu.sync_copy(src_hbm.at[pl.ds(base, CHUNK)], sv)
        pltpu.sync_copy(dst_hbm.at[pl.ds(base, CHUNK)], dv)
        for g in range(CHUNK // 16):
            sl = pl.ds(g * 16, 16)
            pv[sl] = sv[sl] + off
            qv[sl] = dv[sl] + off
        cp_p = pltpu.async_copy(pf_hbm.at[pv], pbuf, sem_p)
        cp_q = pltpu.async_copy(qf_hbm.at[qv], qbuf, sem_q)
        pltpu.sync_copy(r3_hbm.at[c, pl.ds(base, CHUNK)], rbuf)
        cp_p.wait()
        cp_q.wait()

        def row(rw, cr):
            for g in range(HALF // 16):
                sl = pl.ds(g * 16, 16)
                ebuf[rw, sl] = jnp.maximum(
                    pbuf[rw, sl] + qbuf[rw, sl] + rbuf[rw, sl], 0.0)
            return cr

        lax.fori_loop(0, CHUNK, row, 0)
        pltpu.sync_copy(ebuf,
                        enew_hbm.at[pl.ds(base, CHUNK),
                                    pl.ds(c * HALF, HALF)])
        pltpu.sync_copy(ebuf, acc.at[dv], add=True)
        return carry

    lax.fori_loop(0, N_CHUNKS, chunk, 0)
    plsc.subcore_barrier()
    pltpu.sync_copy(acc.at[pl.ds(s * ROWS_PER_TILE, ROWS_PER_TILE)],
                    agg_hbm.at[c, pl.ds(s * ROWS_PER_TILE, ROWS_PER_TILE)])


def _sc_edge_pass(src, dst, p_flat, q_flat, r3, zeros):
    mesh = plsc.VectorSubcoreMesh(core_axis_name="c", subcore_axis_name="s")
    f = pl.kernel(
        _sc_body,
        mesh=mesh,
        out_type=[
            jax.ShapeDtypeStruct((N_EDGES, D_FEAT), jnp.float32),
            jax.ShapeDtypeStruct((2, N_NODES, HALF), jnp.float32),
        ],
        scratch_types=[
            pltpu.VMEM((CHUNK,), jnp.int32),
            pltpu.VMEM((CHUNK,), jnp.int32),
            pltpu.VMEM((CHUNK,), jnp.int32),
            pltpu.VMEM((CHUNK,), jnp.int32),
            pltpu.VMEM((CHUNK, HALF), jnp.float32),
            pltpu.VMEM((CHUNK, HALF), jnp.float32),
            pltpu.VMEM((CHUNK, HALF), jnp.float32),
            pltpu.VMEM((CHUNK, HALF), jnp.float32),
            pltpu.VMEM_SHARED((N_NODES, HALF), jnp.float32),
            pltpu.SemaphoreType.DMA,
            pltpu.SemaphoreType.DMA,
        ],
    )
    return f(src, dst, p_flat, q_flat, r3, zeros)


def kernel(nodes, edges, edge_index, W_e, b_e, W_n, b_n):
    src = edge_index[0]
    dst = edge_index[1]
    we_sd = jnp.stack([W_e[:D_FEAT], W_e[D_FEAT:2 * D_FEAT]])
    we_e = W_e[2 * D_FEAT:]
    wn1 = W_n[:D_FEAT]
    wn2r = W_n[D_FEAT:].reshape(2, HALF, D_FEAT)

    pq = _compute_pq(nodes, we_sd)
    r3 = _compute_r(edges, we_e, b_e)
    p_flat = pq[0].reshape(2 * N_NODES, HALF)
    q_flat = pq[1].reshape(2 * N_NODES, HALF)
    zeros = jnp.zeros((N_NODES, HALF), jnp.float32)

    e_new, agg = _sc_edge_pass(src, dst, p_flat, q_flat, r3, zeros)
    n_new = _node_update(nodes, agg, wn1, wn2r, b_n)
    return (n_new, e_new)


# speedup vs baseline: 2.1564x; 2.1564x over previous
import jax
import jax.numpy as jnp
from jax import lax
from jax.experimental import pallas as pl
from jax.experimental.pallas import tpu as pltpu
from jax.experimental.pallas import tpu_sc as plsc

N_NODES = 10000
N_EDGES = 160000
D_FEAT = 256
HALF = 128

NC = 2
NS = 16
CHUNK = 80
EDGES_PER_TILE = N_EDGES // NS
N_CHUNKS = EDGES_PER_TILE // CHUNK
FLUSH_TILES = 10
FLUSH_ROWS = N_NODES // FLUSH_TILES


def _pq_body(nodes_ref, w_ref, out_ref):
    out_ref[0, 0] = jnp.dot(nodes_ref[...], w_ref[0],
                            preferred_element_type=jnp.float32)


def _compute_pq(nodes, we_sd):
    blk = 2000
    grid = (N_NODES // blk, 2, 2)
    return pl.pallas_call(
        _pq_body,
        grid=grid,
        in_specs=[
            pl.BlockSpec((blk, D_FEAT), lambda i, w, c: (i, 0)),
            pl.BlockSpec((1, D_FEAT, HALF), lambda i, w, c: (w, 0, c)),
        ],
        out_specs=pl.BlockSpec((1, 1, blk, HALF), lambda i, w, c: (w, c, i, 0)),
        out_shape=jax.ShapeDtypeStruct((2, 2, N_NODES, HALF), jnp.float32),
    )(nodes, we_sd)


def _r_body(e_ref, w_ref, b_ref, out_ref):
    out_ref[0] = (jnp.dot(e_ref[...], w_ref[...],
                          preferred_element_type=jnp.float32)
                  + b_ref[...][None, :])


def _compute_r(edges, we_e, b_e):
    blk = 4000
    grid = (N_EDGES // blk, 2)
    return pl.pallas_call(
        _r_body,
        grid=grid,
        in_specs=[
            pl.BlockSpec((blk, 16), lambda j, c: (j, 0)),
            pl.BlockSpec((16, HALF), lambda j, c: (0, c)),
            pl.BlockSpec((HALF,), lambda j, c: (c,)),
        ],
        out_specs=pl.BlockSpec((1, blk, HALF), lambda j, c: (c, j, 0)),
        out_shape=jax.ShapeDtypeStruct((2, N_EDGES, HALF), jnp.float32),
    )(edges, we_e, b_e)


def _node_body(nodes_ref, agg_ref, wn1_ref, wn2_ref, b_ref, out_ref):
    acc = jnp.dot(nodes_ref[...], wn1_ref[...],
                  preferred_element_type=jnp.float32)
    acc += jnp.dot(agg_ref[0], wn2_ref[0], preferred_element_type=jnp.float32)
    acc += jnp.dot(agg_ref[1], wn2_ref[1], preferred_element_type=jnp.float32)
    out_ref[...] = jnp.maximum(acc + b_ref[...][None, :], 0.0)


def _node_update(nodes, agg, wn1, wn2r, b_n):
    blk = 2000
    grid = (N_NODES // blk, 2)
    return pl.pallas_call(
        _node_body,
        grid=grid,
        in_specs=[
            pl.BlockSpec((blk, D_FEAT), lambda i, h: (i, 0)),
            pl.BlockSpec((2, blk, HALF), lambda i, h: (0, i, 0)),
            pl.BlockSpec((D_FEAT, HALF), lambda i, h: (0, h)),
            pl.BlockSpec((2, HALF, HALF), lambda i, h: (0, 0, h)),
            pl.BlockSpec((HALF,), lambda i, h: (h,)),
        ],
        out_specs=pl.BlockSpec((blk, HALF), lambda i, h: (i, h)),
        out_shape=jax.ShapeDtypeStruct((N_NODES, D_FEAT), jnp.float32),
    )(nodes, agg, wn1, wn2r, b_n)


def _sc_body(src_hbm, dst_hbm, pf_hbm, qf_hbm, r3_hbm, z_hbm,
             enew_hbm, agg_hbm,
             sv, dv, pv, qv, pbuf, qbuf, rbuf, ebuf, acc, sem_p, sem_q):
    c = lax.axis_index("c")
    s = lax.axis_index("s")

    @pl.when(s < FLUSH_TILES)
    def _zero():
        pltpu.sync_copy(z_hbm.at[pl.ds(s * FLUSH_ROWS, FLUSH_ROWS)],
                        acc.at[pl.ds(s * FLUSH_ROWS, FLUSH_ROWS)])
    plsc.subcore_barrier()

    off = c * N_NODES

    def chunk(i, carry):
        base = s * EDGES_PER_TILE + i * CHUNK
        pltpu.sync_copy(src_hbm.at[pl.ds(base, CHUNK)], sv)
        pltpu.sync_copy(dst_hbm.at[pl.ds(base, CHUNK)], dv)
        for g in range(CHUNK // 16):
            sl = pl.ds(g * 16, 16)
            pv[sl] = sv[sl] + off
            qv[sl] = dv[sl] + off
        cp_p = pltpu.async_copy(pf_hbm.at[pv], pbuf, sem_p)
        cp_q = pltpu.async_copy(qf_hbm.at[qv], qbuf, sem_q)
        pltpu.sync_copy(r3_hbm.at[c, pl.ds(base, CHUNK)], rbuf)
        cp_p.wait()
        cp_q.wait()

        def row(rw, cr):
            for g in range(HALF // 16):
                sl = pl.ds(g * 16, 16)
                ebuf[rw, sl] = jnp.maximum(
                    pbuf[rw, sl] + qbuf[rw, sl] + rbuf[rw, sl], 0.0)
            return cr

        lax.fori_loop(0, CHUNK, row, 0)
        pltpu.sync_copy(ebuf,
                        enew_hbm.at[pl.ds(base, CHUNK),
                                    pl.ds(c * HALF, HALF)])
        pltpu.sync_copy(ebuf, acc.at[dv], add=True)
        return carry

    lax.fori_loop(0, N_CHUNKS, chunk, 0)
    plsc.subcore_barrier()

    @pl.when(s < FLUSH_TILES)
    def _flush():
        pltpu.sync_copy(acc.at[pl.ds(s * FLUSH_ROWS, FLUSH_ROWS)],
                        agg_hbm.at[c, pl.ds(s * FLUSH_ROWS, FLUSH_ROWS)])


def _sc_edge_pass(src, dst, p_flat, q_flat, r3, zeros):
    mesh = plsc.VectorSubcoreMesh(core_axis_name="c", subcore_axis_name="s")
    f = pl.kernel(
        _sc_body,
        mesh=mesh,
        out_type=[
            jax.ShapeDtypeStruct((N_EDGES, D_FEAT), jnp.float32),
            jax.ShapeDtypeStruct((2, N_NODES, HALF), jnp.float32),
        ],
        scratch_types=[
            pltpu.VMEM((CHUNK,), jnp.int32),
            pltpu.VMEM((CHUNK,), jnp.int32),
            pltpu.VMEM((CHUNK,), jnp.int32),
            pltpu.VMEM((CHUNK,), jnp.int32),
            pltpu.VMEM((CHUNK, HALF), jnp.float32),
            pltpu.VMEM((CHUNK, HALF), jnp.float32),
            pltpu.VMEM((CHUNK, HALF), jnp.float32),
            pltpu.VMEM((CHUNK, HALF), jnp.float32),
            pltpu.VMEM_SHARED((N_NODES, HALF), jnp.float32),
            pltpu.SemaphoreType.DMA,
            pltpu.SemaphoreType.DMA,
        ],
    )
    return f(src, dst, p_flat, q_flat, r3, zeros)


def kernel(nodes, edges, edge_index, W_e, b_e, W_n, b_n):
    src = edge_index[0]
    dst = edge_index[1]
    we_sd = jnp.stack([W_e[:D_FEAT], W_e[D_FEAT:2 * D_FEAT]])
    we_e = W_e[2 * D_FEAT:]
    wn1 = W_n[:D_FEAT]
    wn2r = W_n[D_FEAT:].reshape(2, HALF, D_FEAT)

    pq = _compute_pq(nodes, we_sd)
    r3 = _compute_r(edges, we_e, b_e)
    p_flat = pq[0].reshape(2 * N_NODES, HALF)
    q_flat = pq[1].reshape(2 * N_NODES, HALF)
    zeros = jnp.zeros((N_NODES, HALF), jnp.float32)

    e_new, agg = _sc_edge_pass(src, dst, p_flat, q_flat, r3, zeros)
    n_new = _node_update(nodes, agg, wn1, wn2r, b_n)
    return (n_new, e_new)
